# Initial kernel scaffold; baseline (speedup 1.0000x reference)
import jax
import jax.numpy as jnp
from jax.experimental import pallas as pl


def kernel(state, edge_index, edge_attr, W1, b1, W2, b2):
    raise NotImplementedError("write your pallas kernel here")


# speedup vs baseline: 106.1306x; 106.1306x over previous
import functools

import jax
import jax.numpy as jnp
from jax import lax
from jax.experimental import pallas as pl
from jax.experimental.pallas import tpu as pltpu
from jax.experimental.pallas import tpu_sc as plsc

N_NODES = 10000
N_EDGES = 320000
HIDDEN = 256

NC = 2
NS = 16
NW = NC * NS
L = 16

NPAD = 10240
EPT = N_EDGES // NW

R_TC = 1024
NB = NPAD // R_TC

_mesh = plsc.VectorSubcoreMesh(core_axis_name="c", subcore_axis_name="s")
_sc_params = pltpu.CompilerParams(
    needs_layout_passes=False, use_tc_tiling_on_sc=False)


def _wid():
    return lax.axis_index("c") * NS + lax.axis_index("s")


@functools.partial(
    pl.kernel,
    compiler_params=_sc_params,
    out_type=jax.ShapeDtypeStruct((NW, NPAD), jnp.float32),
    mesh=_mesh,
    scratch_types=[
        pltpu.VMEM((EPT,), jnp.int32),
        pltpu.VMEM((NPAD,), jnp.float32),
    ],
)
def _hist_kernel(dst_hbm, out_hbm, dstv, histv):
    w = _wid()
    zeros = jnp.zeros((L,), jnp.float32)

    def zbody(j, _):
        histv[pl.ds(j * L, L)] = zeros
        return 0

    lax.fori_loop(0, NPAD // L, zbody, 0)
    pltpu.sync_copy(dst_hbm.at[pl.ds(w * EPT, EPT)], dstv)
    ones = jnp.ones((L,), jnp.float32)

    def ebody(i, _):
        idx = dstv[pl.ds(i * L, L)]
        plsc.addupdate_scatter(histv, [idx], ones)
        return 0

    lax.fori_loop(0, EPT // L, ebody, 0)
    pltpu.sync_copy(histv, out_hbm.at[w])


@functools.partial(
    pl.kernel,
    compiler_params=_sc_params,
    out_type=jax.ShapeDtypeStruct((NW, NPAD), jnp.float32),
    mesh=_mesh,
    scratch_types=[
        pltpu.VMEM((NPAD,), jnp.float32),
        pltpu.VMEM((EPT,), jnp.int32),
        pltpu.VMEM((EPT,), jnp.int32),
        pltpu.VMEM((NPAD,), jnp.float32),
    ],
)
def _edge_kernel(src_hbm, dst_hbm, t_hbm, out_hbm, tv, srcv, dstv, gv):
    w = _wid()
    pltpu.sync_copy(t_hbm, tv)
    pltpu.sync_copy(src_hbm.at[pl.ds(w * EPT, EPT)], srcv)
    pltpu.sync_copy(dst_hbm.at[pl.ds(w * EPT, EPT)], dstv)
    zeros = jnp.zeros((L,), jnp.float32)

    def zbody(j, _):
        gv[pl.ds(j * L, L)] = zeros
        return 0

    lax.fori_loop(0, NPAD // L, zbody, 0)

    def ebody(i, _):
        sl = pl.ds(i * L, L)
        si = srcv[sl]
        di = dstv[sl]
        vals = plsc.load_gather(tv, [si])
        plsc.addupdate_scatter(gv, [di], vals)
        return 0

    lax.fori_loop(0, EPT // L, ebody, 0)
    pltpu.sync_copy(gv, out_hbm.at[w])


def _norm_body(hist_ref, s_ref, t_ref, dinv_ref):
    deg = 1.0 + jnp.sum(hist_ref[...], axis=0, keepdims=True)
    dinv = lax.rsqrt(deg)
    dinv_ref[...] = dinv
    t_ref[...] = dinv * s_ref[...]


_norm_tc = pl.pallas_call(
    _norm_body,
    grid=(NB,),
    in_specs=[
        pl.BlockSpec((NW, R_TC), lambda i: (0, i)),
        pl.BlockSpec((1, R_TC), lambda i: (0, i)),
    ],
    out_specs=(
        pl.BlockSpec((1, R_TC), lambda i: (0, i)),
        pl.BlockSpec((1, R_TC), lambda i: (0, i)),
    ),
    out_shape=(
        jax.ShapeDtypeStruct((1, NPAD), jnp.float32),
        jax.ShapeDtypeStruct((1, NPAD), jnp.float32),
    ),
)


def _proj_body(g_ref, t_ref, dinv_ref, w1_ref, b1_ref, w2t_ref, b2_ref,
               x_ref):
    g = jnp.sum(g_ref[...], axis=0, keepdims=True)
    y = dinv_ref[...] * (g + t_ref[...])
    outT = w1_ref[...] * y + b1_ref[...]
    xT = jnp.dot(w2t_ref[...], outT,
                 preferred_element_type=jnp.float32)
    x_ref[...] = xT + b2_ref[...]


_proj_tc = pl.pallas_call(
    _proj_body,
    grid=(NB,),
    in_specs=[
        pl.BlockSpec((NW, R_TC), lambda i: (0, i)),
        pl.BlockSpec((1, R_TC), lambda i: (0, i)),
        pl.BlockSpec((1, R_TC), lambda i: (0, i)),
        pl.BlockSpec((HIDDEN, 1), lambda i: (0, 0)),
        pl.BlockSpec((HIDDEN, 1), lambda i: (0, 0)),
        pl.BlockSpec((1, HIDDEN), lambda i: (0, 0)),
        pl.BlockSpec((1, 1), lambda i: (0, 0)),
    ],
    out_specs=pl.BlockSpec((1, R_TC), lambda i: (0, i)),
    out_shape=jax.ShapeDtypeStruct((1, NPAD), jnp.float32),
)


def kernel(state, edge_index, edge_attr, W1, b1, W2, b2):
    src = edge_index[0]
    dst = edge_index[1]
    s_pad = jnp.pad(state, (0, NPAD - N_NODES))
    hist = _hist_kernel(dst)
    t2, dinv2 = _norm_tc(hist, s_pad.reshape(1, NPAD))
    g = _edge_kernel(src, dst, t2.reshape(NPAD))
    x2 = _proj_tc(g, t2, dinv2,
                  W1.reshape(HIDDEN, 1), b1.reshape(HIDDEN, 1),
                  W2.reshape(1, HIDDEN), b2.reshape(1, 1))
    return x2.reshape(NPAD)[:N_NODES].reshape(N_NODES, 1)


# speedup vs baseline: 114.9132x; 1.0828x over previous
import functools

import jax
import jax.numpy as jnp
from jax import lax
from jax.experimental import pallas as pl
from jax.experimental.pallas import tpu as pltpu
from jax.experimental.pallas import tpu_sc as plsc

N_NODES = 10000
N_EDGES = 320000
HIDDEN = 256

NC = 2
NS = 16
NW = NC * NS
L = 16

NPAD = 10240
EPT = N_EDGES // NW
EPS = N_EDGES // NS
SLOT = NPAD // NS

R_TC = 1024
NB = NPAD // R_TC

_mesh = plsc.VectorSubcoreMesh(core_axis_name="c", subcore_axis_name="s")
_sc_params = pltpu.CompilerParams(
    needs_layout_passes=False, use_tc_tiling_on_sc=False)


def _rsqrt16(x):
    i = plsc.bitcast(x, jnp.int32)
    i = jnp.int32(0x5F3759DF) - (i >> 1)
    y = plsc.bitcast(i, jnp.float32)
    for _ in range(3):
        y = y * (1.5 - 0.5 * x * y * y)
    return y


@functools.partial(
    pl.kernel,
    compiler_params=_sc_params,
    out_type=(
        jax.ShapeDtypeStruct((NC, NPAD), jnp.float32),
        jax.ShapeDtypeStruct((NPAD,), jnp.float32),
        jax.ShapeDtypeStruct((NPAD,), jnp.float32),
    ),
    mesh=_mesh,
    scratch_types=[
        pltpu.VMEM((EPT,), jnp.int32),
        pltpu.VMEM((EPT,), jnp.int32),
        pltpu.VMEM((NPAD,), jnp.float32),
        pltpu.VMEM((NPAD,), jnp.float32),
        pltpu.VMEM((NS, SLOT), jnp.float32),
        pltpu.VMEM((SLOT,), jnp.float32),
        pltpu.VMEM((SLOT,), jnp.float32),
        pltpu.VMEM((SLOT,), jnp.float32),
        pltpu.VMEM_SHARED((NS, NPAD), jnp.float32),
        pltpu.VMEM_SHARED((NPAD,), jnp.float32),
    ],
)
def _sc_kernel(src_hbm, dst_hbm, s_hbm, g_hbm, dinv_hbm, t_hbm,
               idx_a, idx_b, acc, tfull, red, schunk, dchunk, tchunk,
               slots, t_sh):
    c = lax.axis_index("c")
    s = lax.axis_index("s")
    zeros = jnp.zeros((L,), jnp.float32)
    ones = jnp.ones((L,), jnp.float32)

    def zbody(j, _):
        acc[pl.ds(j * L, L)] = zeros
        return 0

    lax.fori_loop(0, NPAD // L, zbody, 0)
    pltpu.sync_copy(dst_hbm.at[pl.ds(s * EPS, EPT)], idx_a)
    pltpu.sync_copy(dst_hbm.at[pl.ds(s * EPS + EPT, EPT)], idx_b)

    def h1body(i, _):
        plsc.addupdate_scatter(acc, [idx_a[pl.ds(i * L, L)]], ones)
        plsc.addupdate_scatter(acc, [idx_b[pl.ds(i * L, L)]], ones)
        return 0

    lax.fori_loop(0, EPT // L, h1body, 0)

    pltpu.sync_copy(acc, slots.at[s])
    plsc.subcore_barrier()
    base = s * SLOT
    pltpu.sync_copy(slots.at[:, pl.ds(base, SLOT)], red)
    pltpu.sync_copy(s_hbm.at[pl.ds(base, SLOT)], schunk)

    def nbody(j, _):
        sl = pl.ds(j * L, L)

        def kbody(k, a):
            return a + red[k, sl]

        deg = lax.fori_loop(0, NS, kbody, jnp.ones((L,), jnp.float32))
        dinv = _rsqrt16(deg)
        dchunk[sl] = dinv
        tchunk[sl] = dinv * schunk[sl]
        return 0

    lax.fori_loop(0, SLOT // L, nbody, 0)
    pltpu.sync_copy(tchunk, t_sh.at[pl.ds(base, SLOT)])

    @pl.when(c == 0)
    def _():
        pltpu.sync_copy(dchunk, dinv_hbm.at[pl.ds(base, SLOT)])
        pltpu.sync_copy(tchunk, t_hbm.at[pl.ds(base, SLOT)])

    plsc.subcore_barrier()
    pltpu.sync_copy(t_sh, tfull)

    def z2body(j, _):
        acc[pl.ds(j * L, L)] = zeros
        return 0

    lax.fori_loop(0, NPAD // L, z2body, 0)
    w = c * NS + s
    pltpu.sync_copy(src_hbm.at[pl.ds(w * EPT, EPT)], idx_a)
    pltpu.sync_copy(dst_hbm.at[pl.ds(w * EPT, EPT)], idx_b)

    def ebody(i, _):
        sl = pl.ds(i * L, L)
        vals = plsc.load_gather(tfull, [idx_a[sl]])
        plsc.addupdate_scatter(acc, [idx_b[sl]], vals)
        return 0

    lax.fori_loop(0, EPT // L, ebody, 0)

    pltpu.sync_copy(acc, slots.at[s])
    plsc.subcore_barrier()
    pltpu.sync_copy(slots.at[:, pl.ds(base, SLOT)], red)

    def gbody(j, _):
        sl = pl.ds(j * L, L)

        def kbody(k, a):
            return a + red[k, sl]

        tot = lax.fori_loop(0, NS, kbody, jnp.zeros((L,), jnp.float32))
        tchunk[sl] = tot
        return 0

    lax.fori_loop(0, SLOT // L, gbody, 0)
    pltpu.sync_copy(tchunk, g_hbm.at[c, pl.ds(base, SLOT)])


def _proj_body(g_ref, t_ref, dinv_ref, w1_ref, b1_ref, w2t_ref, b2_ref,
               x_ref):
    g = jnp.sum(g_ref[...], axis=0, keepdims=True)
    y = dinv_ref[...] * (g + t_ref[...])
    outT = w1_ref[...] * y + b1_ref[...]
    xT = jnp.dot(w2t_ref[...], outT,
                 preferred_element_type=jnp.float32)
    x_ref[...] = xT + b2_ref[...]


_proj_tc = pl.pallas_call(
    _proj_body,
    grid=(NB,),
    in_specs=[
        pl.BlockSpec((NC, R_TC), lambda i: (0, i)),
        pl.BlockSpec((1, R_TC), lambda i: (0, i)),
        pl.BlockSpec((1, R_TC), lambda i: (0, i)),
        pl.BlockSpec((HIDDEN, 1), lambda i: (0, 0)),
        pl.BlockSpec((HIDDEN, 1), lambda i: (0, 0)),
        pl.BlockSpec((1, HIDDEN), lambda i: (0, 0)),
        pl.BlockSpec((1, 1), lambda i: (0, 0)),
    ],
    out_specs=pl.BlockSpec((1, R_TC), lambda i: (0, i)),
    out_shape=jax.ShapeDtypeStruct((1, NPAD), jnp.float32),
)


def kernel(state, edge_index, edge_attr, W1, b1, W2, b2):
    src = edge_index[0]
    dst = edge_index[1]
    s_pad = jnp.pad(state, (0, NPAD - N_NODES))
    g, dinv, t = _sc_kernel(src, dst, s_pad)
    x2 = _proj_tc(g, t.reshape(1, NPAD), dinv.reshape(1, NPAD),
                  W1.reshape(HIDDEN, 1), b1.reshape(HIDDEN, 1),
                  W2.reshape(1, HIDDEN), b2.reshape(1, 1))
    return x2.reshape(NPAD)[:N_NODES].reshape(N_NODES, 1)


# speedup vs baseline: 149.5610x; 1.3015x over previous
import functools

import jax
import jax.numpy as jnp
from jax import lax
from jax.experimental import pallas as pl
from jax.experimental.pallas import tpu as pltpu
from jax.experimental.pallas import tpu_sc as plsc

N_NODES = 10000
N_EDGES = 320000
HIDDEN = 256

NC = 2
NS = 16
NW = NC * NS
L = 16

NPAD = 10240
EPT = N_EDGES // NW
SLOT = NPAD // NS
UE = 5
UZ = 8

R_TC = 1024
NB = NPAD // R_TC

_mesh = plsc.VectorSubcoreMesh(core_axis_name="c", subcore_axis_name="s")
_sc_params = pltpu.CompilerParams(
    needs_layout_passes=False, use_tc_tiling_on_sc=False)


def _rsqrt16(x):
    i = plsc.bitcast(x, jnp.int32)
    i = jnp.int32(0x5F3759DF) - (i >> 1)
    y = plsc.bitcast(i, jnp.float32)
    for _ in range(3):
        y = y * (1.5 - 0.5 * x * y * y)
    return y


@functools.partial(
    pl.kernel,
    compiler_params=_sc_params,
    out_type=(
        jax.ShapeDtypeStruct((NC, NPAD), jnp.float32),
        jax.ShapeDtypeStruct((NPAD,), jnp.float32),
        jax.ShapeDtypeStruct((NPAD,), jnp.float32),
    ),
    mesh=_mesh,
    scratch_types=[
        pltpu.VMEM((EPT,), jnp.int32),
        pltpu.VMEM((EPT,), jnp.int32),
        pltpu.VMEM((EPT,), jnp.int32),
        pltpu.VMEM((NPAD,), jnp.float32),
        pltpu.VMEM((NPAD,), jnp.float32),
        pltpu.VMEM((NS, SLOT), jnp.float32),
        pltpu.VMEM((SLOT,), jnp.float32),
        pltpu.VMEM((SLOT,), jnp.float32),
        pltpu.VMEM((SLOT,), jnp.float32),
        pltpu.VMEM_SHARED((NS, NPAD), jnp.float32),
        pltpu.VMEM_SHARED((NPAD,), jnp.float32),
    ],
)
def _sc_kernel(ei_hbm, s_hbm, g_hbm, dinv_hbm, t_hbm,
               idx_a, idx_b, srcv, acc, tfull, red, schunk, dchunk, tchunk,
               slots, t_sh):
    c = lax.axis_index("c")
    s = lax.axis_index("s")
    zeros = jnp.zeros((L,), jnp.float32)
    ones = jnp.ones((L,), jnp.float32)

    def zero_acc():
        def zbody(j, _):
            for u in range(UZ):
                acc[pl.ds((j * UZ + u) * L, L)] = zeros
            return 0

        lax.fori_loop(0, NPAD // (L * UZ), zbody, 0)

    zero_acc()
    pltpu.sync_copy(ei_hbm.at[1, pl.ds(s * EPT, EPT)], idx_a)
    pltpu.sync_copy(ei_hbm.at[1, pl.ds((s + NS) * EPT, EPT)], idx_b)

    def h1body(i, _):
        for u in range(UE):
            sl = pl.ds((i * UE + u) * L, L)
            plsc.addupdate_scatter(acc, [idx_a[sl]], ones)
            plsc.addupdate_scatter(acc, [idx_b[sl]], ones)
        return 0

    lax.fori_loop(0, EPT // (L * UE), h1body, 0)

    pltpu.sync_copy(acc, slots.at[s])
    plsc.subcore_barrier()
    base = s * SLOT
    pltpu.sync_copy(slots.at[:, pl.ds(base, SLOT)], red)
    pltpu.sync_copy(s_hbm.at[pl.ds(base, SLOT)], schunk)

    def nbody(j, _):
        sl = pl.ds(j * L, L)
        deg = jnp.ones((L,), jnp.float32)
        for k in range(NS):
            deg = deg + red[k, sl]
        dinv = _rsqrt16(deg)
        dchunk[sl] = dinv
        tchunk[sl] = dinv * schunk[sl]
        return 0

    lax.fori_loop(0, SLOT // L, nbody, 0)
    pltpu.sync_copy(tchunk, t_sh.at[pl.ds(base, SLOT)])

    @pl.when(c == 0)
    def _():
        pltpu.sync_copy(dchunk, dinv_hbm.at[pl.ds(base, SLOT)])
        pltpu.sync_copy(tchunk, t_hbm.at[pl.ds(base, SLOT)])

    plsc.subcore_barrier()
    pltpu.sync_copy(t_sh, tfull)

    zero_acc()
    w = c * NS + s
    pltpu.sync_copy(ei_hbm.at[0, pl.ds(w * EPT, EPT)], srcv)

    def edge_sweep(dref):
        def ebody(i, _):
            for u in range(UE):
                sl = pl.ds((i * UE + u) * L, L)
                vals = plsc.load_gather(tfull, [srcv[sl]])
                plsc.addupdate_scatter(acc, [dref[sl]], vals)
            return 0

        lax.fori_loop(0, EPT // (L * UE), ebody, 0)

    @pl.when(c == 0)
    def _():
        edge_sweep(idx_a)

    @pl.when(c == 1)
    def _():
        edge_sweep(idx_b)

    pltpu.sync_copy(acc, slots.at[s])
    plsc.subcore_barrier()
    pltpu.sync_copy(slots.at[:, pl.ds(base, SLOT)], red)

    def gbody(j, _):
        sl = pl.ds(j * L, L)
        tot = jnp.zeros((L,), jnp.float32)
        for k in range(NS):
            tot = tot + red[k, sl]
        tchunk[sl] = tot
        return 0

    lax.fori_loop(0, SLOT // L, gbody, 0)
    pltpu.sync_copy(tchunk, g_hbm.at[c, pl.ds(base, SLOT)])


def _proj_body(g_ref, t_ref, dinv_ref, w1_ref, b1_ref, w2t_ref, b2_ref,
               x_ref):
    g = jnp.sum(g_ref[...], axis=0, keepdims=True)
    y = dinv_ref[...] * (g + t_ref[...])
    outT = w1_ref[...] * y + b1_ref[...]
    xT = jnp.dot(w2t_ref[...], outT,
                 preferred_element_type=jnp.float32)
    x_ref[...] = xT + b2_ref[...]


_proj_tc = pl.pallas_call(
    _proj_body,
    grid=(NB,),
    in_specs=[
        pl.BlockSpec((NC, R_TC), lambda i: (0, i)),
        pl.BlockSpec((1, R_TC), lambda i: (0, i)),
        pl.BlockSpec((1, R_TC), lambda i: (0, i)),
        pl.BlockSpec((HIDDEN, 1), lambda i: (0, 0)),
        pl.BlockSpec((HIDDEN, 1), lambda i: (0, 0)),
        pl.BlockSpec((1, HIDDEN), lambda i: (0, 0)),
        pl.BlockSpec((1, 1), lambda i: (0, 0)),
    ],
    out_specs=pl.BlockSpec((1, R_TC), lambda i: (0, i)),
    out_shape=jax.ShapeDtypeStruct((1, NPAD), jnp.float32),
)


def kernel(state, edge_index, edge_attr, W1, b1, W2, b2):
    s_pad = jnp.pad(state, (0, NPAD - N_NODES))
    g, dinv, t = _sc_kernel(edge_index, s_pad)
    x2 = _proj_tc(g, t.reshape(1, NPAD), dinv.reshape(1, NPAD),
                  W1.reshape(HIDDEN, 1), b1.reshape(HIDDEN, 1),
                  W2.reshape(1, HIDDEN), b2.reshape(1, 1))
    return x2.reshape(NPAD)[:N_NODES].reshape(N_NODES, 1)


# speedup vs baseline: 178.6784x; 1.1947x over previous
import functools

import jax
import jax.numpy as jnp
from jax import lax
from jax.experimental import pallas as pl
from jax.experimental.pallas import tpu as pltpu
from jax.experimental.pallas import tpu_sc as plsc

N_NODES = 10000
N_EDGES = 320000
HIDDEN = 256

NC = 2
NS = 16
NW = NC * NS
L = 16

NPAD = 10240
EPT = N_EDGES // NW
SLOT = NPAD // NS
UE = 5
UZ = 8

R_TC = 1024
NB = NPAD // R_TC

_mesh = plsc.VectorSubcoreMesh(core_axis_name="c", subcore_axis_name="s")
_sc_params = pltpu.CompilerParams(
    needs_layout_passes=False, use_tc_tiling_on_sc=False)


def _rsqrt16(x):
    i = plsc.bitcast(x, jnp.int32)
    i = jnp.int32(0x5F3759DF) - (i >> 1)
    y = plsc.bitcast(i, jnp.float32)
    for _ in range(3):
        y = y * (1.5 - 0.5 * x * y * y)
    return y


@functools.partial(
    pl.kernel,
    compiler_params=_sc_params,
    out_type=(
        jax.ShapeDtypeStruct((NC, NPAD), jnp.float32),
        jax.ShapeDtypeStruct((NPAD,), jnp.float32),
        jax.ShapeDtypeStruct((NPAD,), jnp.float32),
    ),
    mesh=_mesh,
    scratch_types=[
        pltpu.VMEM((EPT,), jnp.int32),
        pltpu.VMEM((EPT,), jnp.int32),
        pltpu.VMEM((EPT,), jnp.int32),
        pltpu.VMEM((NPAD,), jnp.float32),
        pltpu.VMEM((NPAD,), jnp.float32),
        pltpu.VMEM((NS, SLOT), jnp.float32),
        pltpu.VMEM((SLOT,), jnp.float32),
        pltpu.VMEM((SLOT,), jnp.float32),
        pltpu.VMEM((SLOT,), jnp.float32),
        pltpu.VMEM_SHARED((NS, NPAD), jnp.float32),
        pltpu.VMEM_SHARED((NPAD,), jnp.float32),
    ],
)
def _sc_kernel(ei_hbm, s_hbm, g_hbm, dinv_hbm, t_hbm,
               idx_a, idx_b, srcv, acc, tfull, red, schunk, dchunk, tchunk,
               slots, t_sh):
    c = lax.axis_index("c")
    s = lax.axis_index("s")
    zeros = jnp.zeros((L,), jnp.float32)
    ones = jnp.ones((L,), jnp.float32)

    def zero_acc():
        def zbody(j, _):
            for u in range(UZ):
                acc[pl.ds((j * UZ + u) * L, L)] = zeros
            return 0

        lax.fori_loop(0, NPAD // (L * UZ), zbody, 0)

    zero_acc()
    pltpu.sync_copy(ei_hbm.at[1, pl.ds(s * EPT, EPT)], idx_a)
    pltpu.sync_copy(ei_hbm.at[1, pl.ds((s + NS) * EPT, EPT)], idx_b)

    def h1body(i, _):
        sls = [pl.ds((i * UE + u) * L, L) for u in range(UE)]
        ia = [idx_a[sl] for sl in sls]
        ib = [idx_b[sl] for sl in sls]
        for v in ia:
            plsc.addupdate_scatter(acc, [v], ones)
        for v in ib:
            plsc.addupdate_scatter(acc, [v], ones)
        return 0

    lax.fori_loop(0, EPT // (L * UE), h1body, 0)

    pltpu.sync_copy(acc, slots.at[s])
    plsc.subcore_barrier()
    base = s * SLOT
    pltpu.sync_copy(slots.at[:, pl.ds(base, SLOT)], red)
    pltpu.sync_copy(s_hbm.at[pl.ds(base, SLOT)], schunk)

    def nbody(j, _):
        sl = pl.ds(j * L, L)
        deg = jnp.ones((L,), jnp.float32)
        for k in range(NS):
            deg = deg + red[k, sl]
        dinv = _rsqrt16(deg)
        dchunk[sl] = dinv
        tchunk[sl] = dinv * schunk[sl]
        return 0

    lax.fori_loop(0, SLOT // L, nbody, 0)
    pltpu.sync_copy(tchunk, t_sh.at[pl.ds(base, SLOT)])

    @pl.when(c == 0)
    def _():
        pltpu.sync_copy(dchunk, dinv_hbm.at[pl.ds(base, SLOT)])
        pltpu.sync_copy(tchunk, t_hbm.at[pl.ds(base, SLOT)])

    plsc.subcore_barrier()
    pltpu.sync_copy(t_sh, tfull)

    zero_acc()
    w = c * NS + s
    pltpu.sync_copy(ei_hbm.at[0, pl.ds(w * EPT, EPT)], srcv)

    def edge_sweep(dref):
        def ebody(i, _):
            sls = [pl.ds((i * UE + u) * L, L) for u in range(UE)]
            sv = [srcv[sl] for sl in sls]
            dv = [dref[sl] for sl in sls]
            gv = [plsc.load_gather(tfull, [x]) for x in sv]
            for u in range(UE):
                plsc.addupdate_scatter(acc, [dv[u]], gv[u])
            return 0

        lax.fori_loop(0, EPT // (L * UE), ebody, 0)

    @pl.when(c == 0)
    def _():
        edge_sweep(idx_a)

    @pl.when(c == 1)
    def _():
        edge_sweep(idx_b)

    pltpu.sync_copy(acc, slots.at[s])
    plsc.subcore_barrier()
    pltpu.sync_copy(slots.at[:, pl.ds(base, SLOT)], red)

    def gbody(j, _):
        sl = pl.ds(j * L, L)
        tot = jnp.zeros((L,), jnp.float32)
        for k in range(NS):
            tot = tot + red[k, sl]
        tchunk[sl] = tot
        return 0

    lax.fori_loop(0, SLOT // L, gbody, 0)
    pltpu.sync_copy(tchunk, g_hbm.at[c, pl.ds(base, SLOT)])


def _proj_body(g_ref, t_ref, dinv_ref, w1_ref, b1_ref, w2t_ref, b2_ref,
               x_ref):
    g = jnp.sum(g_ref[...], axis=0, keepdims=True)
    y = dinv_ref[...] * (g + t_ref[...])
    outT = w1_ref[...] * y + b1_ref[...]
    xT = jnp.dot(w2t_ref[...], outT,
                 preferred_element_type=jnp.float32)
    x_ref[...] = xT + b2_ref[...]


_proj_tc = pl.pallas_call(
    _proj_body,
    grid=(NB,),
    in_specs=[
        pl.BlockSpec((NC, R_TC), lambda i: (0, i)),
        pl.BlockSpec((1, R_TC), lambda i: (0, i)),
        pl.BlockSpec((1, R_TC), lambda i: (0, i)),
        pl.BlockSpec((HIDDEN, 1), lambda i: (0, 0)),
        pl.BlockSpec((HIDDEN, 1), lambda i: (0, 0)),
        pl.BlockSpec((1, HIDDEN), lambda i: (0, 0)),
        pl.BlockSpec((1, 1), lambda i: (0, 0)),
    ],
    out_specs=pl.BlockSpec((1, R_TC), lambda i: (0, i)),
    out_shape=jax.ShapeDtypeStruct((1, NPAD), jnp.float32),
)


def kernel(state, edge_index, edge_attr, W1, b1, W2, b2):
    s_pad = jnp.pad(state, (0, NPAD - N_NODES))
    g, dinv, t = _sc_kernel(edge_index, s_pad)
    x2 = _proj_tc(g, t.reshape(1, NPAD), dinv.reshape(1, NPAD),
                  W1.reshape(HIDDEN, 1), b1.reshape(HIDDEN, 1),
                  W2.reshape(1, HIDDEN), b2.reshape(1, 1))
    return x2.reshape(NPAD)[:N_NODES].reshape(N_NODES, 1)


# speedup vs baseline: 190.1317x; 1.0641x over previous
import functools

import jax
import jax.numpy as jnp
from jax import lax
from jax.experimental import pallas as pl
from jax.experimental.pallas import tpu as pltpu
from jax.experimental.pallas import tpu_sc as plsc

N_NODES = 10000
N_EDGES = 320000
HIDDEN = 256

NC = 2
NS = 16
NW = NC * NS
L = 16

NPAD = 10240
EPT = N_EDGES // NW
SLOT = NPAD // NS
UE = 5
UZ = 8

R_TC = 2048
NB = NPAD // R_TC

_mesh = plsc.VectorSubcoreMesh(core_axis_name="c", subcore_axis_name="s")
_sc_params = pltpu.CompilerParams(
    needs_layout_passes=False, use_tc_tiling_on_sc=False)


def _rsqrt16(x):
    i = plsc.bitcast(x, jnp.int32)
    i = jnp.int32(0x5F3759DF) - (i >> 1)
    y = plsc.bitcast(i, jnp.float32)
    for _ in range(3):
        y = y * (1.5 - 0.5 * x * y * y)
    return y


@functools.partial(
    pl.kernel,
    compiler_params=_sc_params,
    out_type=(
        jax.ShapeDtypeStruct((NC, NPAD), jnp.float32),
        jax.ShapeDtypeStruct((1, NPAD), jnp.float32),
        jax.ShapeDtypeStruct((1, NPAD), jnp.float32),
    ),
    mesh=_mesh,
    scratch_types=[
        pltpu.VMEM((EPT,), jnp.int32),
        pltpu.VMEM((EPT,), jnp.int32),
        pltpu.VMEM((EPT,), jnp.int32),
        pltpu.VMEM((NPAD,), jnp.float32),
        pltpu.VMEM((NPAD,), jnp.float32),
        pltpu.VMEM((NS, SLOT), jnp.float32),
        pltpu.VMEM((SLOT,), jnp.float32),
        pltpu.VMEM((SLOT,), jnp.float32),
        pltpu.VMEM((SLOT,), jnp.float32),
        pltpu.VMEM_SHARED((NS, NPAD), jnp.float32),
        pltpu.VMEM_SHARED((NPAD,), jnp.float32),
    ],
)
def _sc_kernel(ei_hbm, s_hbm, g_hbm, dinv_hbm, t_hbm,
               idx_a, idx_b, srcv, acc, tfull, red, schunk, dchunk, tchunk,
               slots, t_sh):
    c = lax.axis_index("c")
    s = lax.axis_index("s")
    zeros = jnp.zeros((L,), jnp.float32)
    ones = jnp.ones((L,), jnp.float32)

    def zero_acc():
        def zbody(j, _):
            for u in range(UZ):
                acc[pl.ds((j * UZ + u) * L, L)] = zeros
            return 0

        lax.fori_loop(0, NPAD // (L * UZ), zbody, 0)

    zero_acc()
    pltpu.sync_copy(ei_hbm.at[1, pl.ds(s * EPT, EPT)], idx_a)
    pltpu.sync_copy(ei_hbm.at[1, pl.ds((s + NS) * EPT, EPT)], idx_b)

    def h1body(i, _):
        sls = [pl.ds((i * UE + u) * L, L) for u in range(UE)]
        ia = [idx_a[sl] for sl in sls]
        ib = [idx_b[sl] for sl in sls]
        for v in ia:
            plsc.addupdate_scatter(acc, [v], ones)
        for v in ib:
            plsc.addupdate_scatter(acc, [v], ones)
        return 0

    lax.fori_loop(0, EPT // (L * UE), h1body, 0)

    pltpu.sync_copy(acc, slots.at[s])
    plsc.subcore_barrier()
    base = s * SLOT
    pltpu.sync_copy(slots.at[:, pl.ds(base, SLOT)], red)
    pltpu.sync_copy(s_hbm.at[pl.ds(base, SLOT)], schunk)

    def nbody(j, _):
        sl = pl.ds(j * L, L)
        deg = jnp.ones((L,), jnp.float32)
        for k in range(NS):
            deg = deg + red[k, sl]
        dinv = _rsqrt16(deg)
        dchunk[sl] = dinv
        tchunk[sl] = dinv * schunk[sl]
        return 0

    lax.fori_loop(0, SLOT // L, nbody, 0)
    pltpu.sync_copy(tchunk, t_sh.at[pl.ds(base, SLOT)])

    @pl.when(c == 0)
    def _():
        pltpu.sync_copy(dchunk, dinv_hbm.at[0, pl.ds(base, SLOT)])
        pltpu.sync_copy(tchunk, t_hbm.at[0, pl.ds(base, SLOT)])

    plsc.subcore_barrier()
    pltpu.sync_copy(t_sh, tfull)

    zero_acc()
    w = c * NS + s
    pltpu.sync_copy(ei_hbm.at[0, pl.ds(w * EPT, EPT)], srcv)

    def edge_sweep(dref):
        def ebody(i, _):
            sls = [pl.ds((i * UE + u) * L, L) for u in range(UE)]
            sv = [srcv[sl] for sl in sls]
            dv = [dref[sl] for sl in sls]
            gv = [plsc.load_gather(tfull, [x]) for x in sv]
            for u in range(UE):
                plsc.addupdate_scatter(acc, [dv[u]], gv[u])
            return 0

        lax.fori_loop(0, EPT // (L * UE), ebody, 0)

    @pl.when(c == 0)
    def _():
        edge_sweep(idx_a)

    @pl.when(c == 1)
    def _():
        edge_sweep(idx_b)

    pltpu.sync_copy(acc, slots.at[s])
    plsc.subcore_barrier()
    pltpu.sync_copy(slots.at[:, pl.ds(base, SLOT)], red)

    def gbody(j, _):
        sl = pl.ds(j * L, L)
        tot = jnp.zeros((L,), jnp.float32)
        for k in range(NS):
            tot = tot + red[k, sl]
        tchunk[sl] = tot
        return 0

    lax.fori_loop(0, SLOT // L, gbody, 0)
    pltpu.sync_copy(tchunk, g_hbm.at[c, pl.ds(base, SLOT)])


def _proj_body(g_ref, t_ref, dinv_ref, w1_ref, b1_ref, w2t_ref, b2_ref,
               x_ref):
    g = jnp.sum(g_ref[...], axis=0, keepdims=True)
    y = dinv_ref[...] * (g + t_ref[...])
    outT = w1_ref[...] * y + b1_ref[...]
    xT = jnp.dot(w2t_ref[...], outT,
                 preferred_element_type=jnp.float32)
    x_ref[...] = xT + b2_ref[...]


_proj_tc = pl.pallas_call(
    _proj_body,
    grid=(NB,),
    in_specs=[
        pl.BlockSpec((NC, R_TC), lambda i: (0, i)),
        pl.BlockSpec((1, R_TC), lambda i: (0, i)),
        pl.BlockSpec((1, R_TC), lambda i: (0, i)),
        pl.BlockSpec((HIDDEN, 1), lambda i: (0, 0)),
        pl.BlockSpec((HIDDEN, 1), lambda i: (0, 0)),
        pl.BlockSpec((1, HIDDEN), lambda i: (0, 0)),
        pl.BlockSpec((1, 1), lambda i: (0, 0)),
    ],
    out_specs=pl.BlockSpec((1, R_TC), lambda i: (0, i)),
    out_shape=jax.ShapeDtypeStruct((1, NPAD), jnp.float32),
)


def kernel(state, edge_index, edge_attr, W1, b1, W2, b2):
    s_pad = jnp.pad(state, (0, NPAD - N_NODES))
    g, dinv, t = _sc_kernel(edge_index, s_pad)
    x2 = _proj_tc(g, t, dinv,
                  W1.reshape(HIDDEN, 1), b1.reshape(HIDDEN, 1),
                  W2.reshape(1, HIDDEN), b2.reshape(1, 1))
    return x2.reshape(NPAD)[:N_NODES].reshape(N_NODES, 1)
